# Initial kernel scaffold; baseline (speedup 1.0000x reference)
import jax
import jax.numpy as jnp
from jax.experimental import pallas as pl


def kernel(proposals, objectness, image_shapes):
    raise NotImplementedError("write your pallas kernel here")


# speedup vs baseline: 9.6371x; 9.6371x over previous
import functools
import jax
import numpy as np
import jax.numpy as jnp
from jax import lax
from jax.experimental import pallas as pl
from jax.experimental.pallas import tpu as pltpu
from jax.experimental.pallas import tpu_sc as plsc

NIMG = 4
N = 20000
NV = N // 16
K = 1000
KP = 1008
KV = KP // 16
OUTCAP = 1024
NMS_T = 0.7
MIN_SIZE = 1e-3
MINI32 = np.int32(-2147483648)
NEG1 = np.int32(-1)

_mesh = plsc.VectorSubcoreMesh(core_axis_name="c", subcore_axis_name="s",
                               num_cores=2, num_subcores=16)


def _popcnt(mask):
  return plsc.all_reduce_population_count(mask)[0]


def _select_pass(shift, cnt, q, out_off, src_k, src_i, dst_k, dst_i,
                 out_k, out_i, hist, iota, final):
  for h in range(16):
    hist[pl.ds(h * 16, 16)] = jnp.zeros((16,), jnp.int32)

  nvreg = (cnt + 15) // 16

  def hbody(j, carry):
    off = j * 16
    kv = src_k[pl.ds(off, 16)]
    valid = (off + iota) < cnt
    byte = lax.shift_right_logical(kv, shift) & 0xFF
    c16, last = plsc.scan_count(byte, mask=valid)
    plsc.addupdate_scatter(hist, [byte], c16, mask=last)
    return carry

  lax.fori_loop(0, nvreg, hbody, 0)

  def sbody(t, carry):
    cum, B, cumAb = carry
    vb = 15 - t
    row = hist[pl.ds(vb * 16, 16)]
    rrow = lax.rev(row, (0,))
    incl = plsc.cumsum(rrow)
    excl = cum + incl - rrow
    cond = (excl < q) & ((excl + rrow) >= q)
    bids = vb * 16 + 15 - iota
    B = B + jnp.sum(jnp.where(cond, bids, 0))
    cumAb = cumAb + jnp.sum(jnp.where(cond, excl, 0))
    return cum + incl[15], B, cumAb

  _, B, cumAb = lax.fori_loop(0, 16, sbody, (jnp.int32(0), jnp.int32(0),
                                             jnp.int32(0)))
  qrem = q - cumAb

  def cbody(j, carry):
    o_off, c_off, taken = carry
    off = j * 16
    kv = src_k[pl.ds(off, 16)]
    iv = (off + iota) if src_i is None else src_i[pl.ds(off, 16)]
    valid = (off + iota) < cnt
    byte = lax.shift_right_logical(kv, shift) & 0xFF
    mdef = valid & (byte > B)
    plsc.store_compressed(out_k.at[pl.ds(o_off, 16)], kv, mask=mdef)
    plsc.store_compressed(out_i.at[pl.ds(o_off, 16)], iv, mask=mdef)
    o_off = o_off + _popcnt(mdef)
    mc = valid & (byte == B)
    if final:
      pc = plsc.cumsum(jnp.where(mc, 1, 0).astype(jnp.int32))
      mtake = mc & ((taken + pc) <= qrem)
      plsc.store_compressed(out_k.at[pl.ds(o_off, 16)], kv, mask=mtake)
      plsc.store_compressed(out_i.at[pl.ds(o_off, 16)], iv, mask=mtake)
      nt = _popcnt(mtake)
      return o_off + nt, c_off, taken + nt
    else:
      plsc.store_compressed(dst_k.at[pl.ds(c_off, 16)], kv, mask=mc)
      plsc.store_compressed(dst_i.at[pl.ds(c_off, 16)], iv, mask=mc)
      return o_off, c_off + _popcnt(mc), taken

  out_off, c_off, _ = lax.fori_loop(0, nvreg, cbody,
                                    (out_off, jnp.int32(0), jnp.int32(0)))
  return c_off, qrem, out_off


def _sort_pass(shift, src_k, src_i, dst_k, dst_i, hist, start, iota):
  for h in range(16):
    hist[pl.ds(h * 16, 16)] = jnp.zeros((16,), jnp.int32)

  def hbody(j, carry):
    kv = src_k[pl.ds(j * 16, 16)]
    byte = lax.shift_right_logical(kv, shift) & 0xFF
    c16, last = plsc.scan_count(byte)
    plsc.addupdate_scatter(hist, [byte], c16, mask=last)
    return carry

  lax.fori_loop(0, KV, hbody, 0)

  def obody(t, cum):
    vb = 15 - t
    row = hist[pl.ds(vb * 16, 16)]
    rrow = lax.rev(row, (0,))
    incl = plsc.cumsum(rrow)
    excl = cum + incl - rrow
    start[pl.ds(vb * 16, 16)] = lax.rev(excl, (0,))
    return cum + incl[15]

  lax.fori_loop(0, 16, obody, jnp.int32(0))

  def pbody(j, carry):
    kv = src_k[pl.ds(j * 16, 16)]
    iv = src_i[pl.ds(j * 16, 16)]
    byte = lax.shift_right_logical(kv, shift) & 0xFF
    c16, last = plsc.scan_count(byte)
    base = plsc.load_gather(start, [byte])
    pos = base + c16 - 1
    plsc.store_scatter(dst_k, [pos], kv)
    plsc.store_scatter(dst_i, [pos], iv)
    plsc.addupdate_scatter(start, [byte], c16, mask=last)
    return carry

  lax.fori_loop(0, KV, pbody, 0)


def _iou_gt(cx1, cy1, cx2, cy2, car, ox1, oy1, ox2, oy2, oar):
  ltx = jnp.maximum(cx1, ox1)
  lty = jnp.maximum(cy1, oy1)
  rbx = jnp.minimum(cx2, ox2)
  rby = jnp.minimum(cy2, oy2)
  ww = jnp.maximum(rbx - ltx, 0.0)
  hh = jnp.maximum(rby - lty, 0.0)
  inter = ww * hh
  union = car + oar - inter
  return (inter / (union + 1e-9)) > NMS_T


def _rpn_body(prop, obj, shapes, boxes_out, scores_out,
              stage, key_a, idx_a, key_b, idx_b,
              out_k, out_i, out_k2, out_i2, hist, start,
              x1, y1, x2, y2, area, ssc, keep, ckb,
              fb, fs, shp, sem):
  wid = lax.axis_index("s") * 2 + lax.axis_index("c")
  img = wid
  iota = lax.iota(jnp.int32, 16)

  @pl.when(wid < NIMG)
  def _():
    pltpu.sync_copy(obj.at[img], stage)

    def kbody(j, carry):
      sv = stage[pl.ds(j * 16, 16)]
      b = plsc.bitcast(sv, jnp.int32)
      key = jnp.where(b < 0, b ^ NEG1, b ^ MINI32)
      key_a[pl.ds(j * 16, 16)] = key
      return carry

    lax.fori_loop(0, NV, kbody, 0)

    for z in range(OUTCAP // 16):
      out_k[pl.ds(z * 16, 16)] = jnp.zeros((16,), jnp.int32)
      out_i[pl.ds(z * 16, 16)] = jnp.zeros((16,), jnp.int32)

    cnt = jnp.int32(N)
    q = jnp.int32(K)
    ooff = jnp.int32(0)
    cnt, q, ooff = _select_pass(24, cnt, q, ooff, key_a, None, key_b, idx_b,
                                out_k, out_i, hist, iota, final=False)
    cnt, q, ooff = _select_pass(16, cnt, q, ooff, key_b, idx_b, key_a, idx_a,
                                out_k, out_i, hist, iota, final=False)
    cnt, q, ooff = _select_pass(8, cnt, q, ooff, key_a, idx_a, key_b, idx_b,
                                out_k, out_i, hist, iota, final=False)
    cnt, q, ooff = _select_pass(0, cnt, q, ooff, key_b, idx_b, None, None,
                                out_k, out_i, hist, iota, final=True)

    _sort_pass(0, out_k, out_i, out_k2, out_i2, hist, start, iota)
    _sort_pass(8, out_k2, out_i2, out_k, out_i, hist, start, iota)
    _sort_pass(16, out_k, out_i, out_k2, out_i2, hist, start, iota)
    _sort_pass(24, out_k2, out_i2, out_k, out_i, hist, start, iota)

    planes = (x1, y1, x2, y2)
    handles = []
    for cd in range(4):
      for c in range(8):
        handles.append(pltpu.async_copy(
            prop.at[img].at[cd].at[out_i.at[pl.ds(c * 128, 128)]],
            planes[cd].at[pl.ds(c * 128, 128)], sem))
    for h in handles:
      h.wait()

    shw = plsc.load_gather(shp, [jnp.full((16,), img, jnp.int32), iota & 1])
    hf = shw[0].astype(jnp.float32)
    wf = shw[1].astype(jnp.float32)

    def vbody(v, carry):
      o = v * 16
      x1v = x1[pl.ds(o, 16)]
      y1v = y1[pl.ds(o, 16)]
      x2v = x2[pl.ds(o, 16)]
      y2v = y2[pl.ds(o, 16)]
      kv = out_k[pl.ds(v * 16, 16)]
      bb = jnp.where(kv >= 0, kv ^ NEG1, kv ^ MINI32)
      sc = plsc.bitcast(bb, jnp.float32)
      x1c = jnp.minimum(jnp.maximum(x1v, 0.0), wf)
      y1c = jnp.minimum(jnp.maximum(y1v, 0.0), hf)
      x2c = jnp.minimum(jnp.maximum(x2v, 0.0), wf)
      y2c = jnp.minimum(jnp.maximum(y2v, 0.0), hf)
      ws = x2c - x1c
      hs = y2c - y1c
      valid = (ws >= MIN_SIZE) & (hs >= MIN_SIZE) & (sc >= 0.0)
      x1[pl.ds(o, 16)] = x1c
      y1[pl.ds(o, 16)] = y1c
      x2[pl.ds(o, 16)] = x2c
      y2[pl.ds(o, 16)] = y2c
      area[pl.ds(o, 16)] = ws * hs
      ssc[pl.ds(o, 16)] = sc
      keep[pl.ds(o, 16)] = jnp.where(valid, 1, 0).astype(jnp.int32)
      return carry

    lax.fori_loop(0, KV, vbody, 0)

    def nbody(b, carry):
      base = b * 16
      bx1 = x1[pl.ds(base, 16)]
      by1 = y1[pl.ds(base, 16)]
      bx2 = x2[pl.ds(base, 16)]
      by2 = y2[pl.ds(base, 16)]
      bar = area[pl.ds(base, 16)]
      kv = keep[pl.ds(base, 16)]
      for i in range(16):
        gt = _iou_gt(bx1[i], by1[i], bx2[i], by2[i], bar[i],
                     bx1, by1, bx2, by2, bar)
        sup = gt & (iota > i) & (kv[i] != 0)
        kv = jnp.where(sup, 0, kv)
      keep[pl.ds(base, 16)] = kv
      mask = kv != 0
      plsc.store_compressed(ckb.at[pl.ds(0, 16)], bx1, mask=mask)
      plsc.store_compressed(ckb.at[pl.ds(16, 16)], by1, mask=mask)
      plsc.store_compressed(ckb.at[pl.ds(32, 16)], bx2, mask=mask)
      plsc.store_compressed(ckb.at[pl.ds(48, 16)], by2, mask=mask)
      plsc.store_compressed(ckb.at[pl.ds(64, 16)], bar, mask=mask)
      nk = _popcnt(mask)

      def tbody(t, carry2):
        tt = jnp.full((16,), t, jnp.int32)
        cx1 = plsc.load_gather(ckb, [tt])
        cy1 = plsc.load_gather(ckb, [tt + 16])
        cx2 = plsc.load_gather(ckb, [tt + 32])
        cy2 = plsc.load_gather(ckb, [tt + 48])
        car = plsc.load_gather(ckb, [tt + 64])

        def wbody(v, carry3):
          o = v * 16
          gt = _iou_gt(cx1, cy1, cx2, cy2, car,
                       x1[pl.ds(o, 16)], y1[pl.ds(o, 16)],
                       x2[pl.ds(o, 16)], y2[pl.ds(o, 16)],
                       area[pl.ds(o, 16)])
          keep[pl.ds(o, 16)] = jnp.where(gt, 0, keep[pl.ds(o, 16)])
          return carry3

        lax.fori_loop(b + 1, KV, wbody, 0)
        return carry2

      lax.fori_loop(0, nk, tbody, 0)
      return carry

    lax.fori_loop(0, KV, nbody, 0)

    for z in range(KV):
      fs[pl.ds(z * 16, 16)] = jnp.full((16,), -1e9, jnp.float32)
    for z in range(KP * 4 // 16):
      fb[pl.ds(z * 16, 16)] = jnp.zeros((16,), jnp.float32)

    def fbody(v, off):
      o = v * 16
      mask = keep[pl.ds(o, 16)] != 0
      pc = plsc.cumsum(jnp.where(mask, 1, 0).astype(jnp.int32))
      pos = off + pc - 1
      plsc.store_scatter(fs, [pos], ssc[pl.ds(o, 16)], mask=mask)
      plsc.store_scatter(fb, [pos * 4], x1[pl.ds(o, 16)], mask=mask)
      plsc.store_scatter(fb, [pos * 4 + 1], y1[pl.ds(o, 16)], mask=mask)
      plsc.store_scatter(fb, [pos * 4 + 2], x2[pl.ds(o, 16)], mask=mask)
      plsc.store_scatter(fb, [pos * 4 + 3], y2[pl.ds(o, 16)], mask=mask)
      return off + _popcnt(mask)

    lax.fori_loop(0, KV, fbody, jnp.int32(0))

    pltpu.sync_copy(fb.at[pl.ds(0, 4 * K)], boxes_out.at[img])
    pltpu.sync_copy(fs.at[pl.ds(0, K)], scores_out.at[img])


@functools.partial(
    pl.kernel,
    out_type=(jax.ShapeDtypeStruct((NIMG, 4 * K), jnp.float32),
              jax.ShapeDtypeStruct((NIMG, K), jnp.float32)),
    mesh=_mesh,
    compiler_params=pltpu.CompilerParams(needs_layout_passes=False,
                                         use_tc_tiling_on_sc=False),
    scratch_types=[
        pltpu.VMEM((N,), jnp.float32),
        pltpu.VMEM((N,), jnp.int32),
        pltpu.VMEM((N,), jnp.int32),
        pltpu.VMEM((N,), jnp.int32),
        pltpu.VMEM((N,), jnp.int32),
        pltpu.VMEM((OUTCAP,), jnp.int32),
        pltpu.VMEM((OUTCAP,), jnp.int32),
        pltpu.VMEM((OUTCAP,), jnp.int32),
        pltpu.VMEM((OUTCAP,), jnp.int32),
        pltpu.VMEM((256,), jnp.int32),
        pltpu.VMEM((256,), jnp.int32),
        pltpu.VMEM((OUTCAP,), jnp.float32),
        pltpu.VMEM((OUTCAP,), jnp.float32),
        pltpu.VMEM((OUTCAP,), jnp.float32),
        pltpu.VMEM((OUTCAP,), jnp.float32),
        pltpu.VMEM((KP,), jnp.float32),
        pltpu.VMEM((KP,), jnp.float32),
        pltpu.VMEM((KP,), jnp.int32),
        pltpu.VMEM((80,), jnp.float32),
        pltpu.VMEM((KP * 4,), jnp.float32),
        pltpu.VMEM((KP,), jnp.float32),
        pltpu.VMEM((NIMG, 2), jnp.int32),
        pltpu.SemaphoreType.DMA,
    ],
)
def _rpn_call(prop, obj, shapes, boxes_out, scores_out, *scratch):
  (stage, key_a, idx_a, key_b, idx_b, out_k, out_i, out_k2, out_i2,
   hist, start, x1, y1, x2, y2, area, ssc, keep, ckb, fb, fs,
   shp, sem) = scratch
  pltpu.sync_copy(shapes, shp)
  _rpn_body(prop, obj, shapes, boxes_out, scores_out,
            stage, key_a, idx_a, key_b, idx_b,
            out_k, out_i, out_k2, out_i2, hist, start,
            x1, y1, x2, y2, area, ssc, keep, ckb, fb, fs, shp, sem)


@jax.jit
def kernel(proposals, objectness, image_shapes):
  planar = jnp.transpose(proposals, (0, 2, 1))
  fb, fs = _rpn_call(planar, objectness, image_shapes)
  return fb.reshape(NIMG, K, 4), fs


# speedup vs baseline: 12.1908x; 1.2650x over previous
import functools
import jax
import numpy as np
import jax.numpy as jnp
from jax import lax
from jax.experimental import pallas as pl
from jax.experimental.pallas import tpu as pltpu
from jax.experimental.pallas import tpu_sc as plsc

NIMG = 4
N = 20000
NV = N // 16
K = 1000
KP = 1008
KV = KP // 16
OUTCAP = 1024
NMS_T = 0.7
MIN_SIZE = 1e-3
MINI32 = np.int32(-2147483648)
NEG1 = np.int32(-1)

_mesh = plsc.VectorSubcoreMesh(core_axis_name="c", subcore_axis_name="s",
                               num_cores=2, num_subcores=16)


def _popcnt(mask):
  return plsc.all_reduce_population_count(mask)[0]


def _select_pass0(q, stage, key_a, out_k, out_i, dst_k, dst_i, hist, iota):
  for h in range(16):
    hist[pl.ds(h * 16, 16)] = jnp.zeros((16,), jnp.int32)

  def hbody(j, carry):
    for u in range(2):
      off = j * 32 + u * 16
      sv = stage[pl.ds(off, 16)]
      b = plsc.bitcast(sv, jnp.int32)
      key = jnp.where(b < 0, b ^ NEG1, b ^ MINI32)
      key_a[pl.ds(off, 16)] = key
      byte = lax.shift_right_logical(key, 24) & 0xFF
      c16, last = plsc.scan_count(byte)
      plsc.addupdate_scatter(hist, [byte], c16, mask=last)
    return carry

  lax.fori_loop(0, NV // 2, hbody, 0)

  def sbody(t, carry):
    cum, B, cumAb = carry
    vb = 15 - t
    row = hist[pl.ds(vb * 16, 16)]
    rrow = lax.rev(row, (0,))
    incl = plsc.cumsum(rrow)
    excl = cum + incl - rrow
    cond = (excl < q) & ((excl + rrow) >= q)
    bids = vb * 16 + 15 - iota
    B = B + jnp.sum(jnp.where(cond, bids, 0))
    cumAb = cumAb + jnp.sum(jnp.where(cond, excl, 0))
    return cum + incl[15], B, cumAb

  _, B, cumAb = lax.fori_loop(0, 16, sbody, (jnp.int32(0), jnp.int32(0),
                                             jnp.int32(0)))

  def cbody(j, carry):
    o_off, c_off = carry
    off = j * 16
    kv = key_a[pl.ds(off, 16)]
    iv = off + iota
    byte = lax.shift_right_logical(kv, 24) & 0xFF
    mdef = byte > B
    plsc.store_compressed(out_k.at[pl.ds(o_off, 16)], kv, mask=mdef)
    plsc.store_compressed(out_i.at[pl.ds(o_off, 16)], iv, mask=mdef)
    o_off = o_off + _popcnt(mdef)
    mc = byte == B
    plsc.store_compressed(dst_k.at[pl.ds(c_off, 16)], kv, mask=mc)
    plsc.store_compressed(dst_i.at[pl.ds(c_off, 16)], iv, mask=mc)
    return o_off, c_off + _popcnt(mc)

  out_off, c_off = lax.fori_loop(0, NV, cbody, (jnp.int32(0), jnp.int32(0)))
  return c_off, q - cumAb, out_off


def _select_pass(shift, cnt, q, out_off, src_k, src_i, dst_k, dst_i,
                 out_k, out_i, hist, iota, final):
  for h in range(16):
    hist[pl.ds(h * 16, 16)] = jnp.zeros((16,), jnp.int32)

  nvreg = (cnt + 15) // 16

  def hbody(j, carry):
    off = j * 16
    kv = src_k[pl.ds(off, 16)]
    valid = (off + iota) < cnt
    byte = lax.shift_right_logical(kv, shift) & 0xFF
    c16, last = plsc.scan_count(byte, mask=valid)
    plsc.addupdate_scatter(hist, [byte], c16, mask=last)
    return carry

  lax.fori_loop(0, nvreg, hbody, 0)

  def sbody(t, carry):
    cum, B, cumAb = carry
    vb = 15 - t
    row = hist[pl.ds(vb * 16, 16)]
    rrow = lax.rev(row, (0,))
    incl = plsc.cumsum(rrow)
    excl = cum + incl - rrow
    cond = (excl < q) & ((excl + rrow) >= q)
    bids = vb * 16 + 15 - iota
    B = B + jnp.sum(jnp.where(cond, bids, 0))
    cumAb = cumAb + jnp.sum(jnp.where(cond, excl, 0))
    return cum + incl[15], B, cumAb

  _, B, cumAb = lax.fori_loop(0, 16, sbody, (jnp.int32(0), jnp.int32(0),
                                             jnp.int32(0)))
  qrem = q - cumAb

  def cbody(j, carry):
    o_off, c_off, taken = carry
    off = j * 16
    kv = src_k[pl.ds(off, 16)]
    iv = (off + iota) if src_i is None else src_i[pl.ds(off, 16)]
    valid = (off + iota) < cnt
    byte = lax.shift_right_logical(kv, shift) & 0xFF
    mdef = valid & (byte > B)
    plsc.store_compressed(out_k.at[pl.ds(o_off, 16)], kv, mask=mdef)
    plsc.store_compressed(out_i.at[pl.ds(o_off, 16)], iv, mask=mdef)
    o_off = o_off + _popcnt(mdef)
    mc = valid & (byte == B)
    if final:
      pc = plsc.cumsum(jnp.where(mc, 1, 0).astype(jnp.int32))
      mtake = mc & ((taken + pc) <= qrem)
      plsc.store_compressed(out_k.at[pl.ds(o_off, 16)], kv, mask=mtake)
      plsc.store_compressed(out_i.at[pl.ds(o_off, 16)], iv, mask=mtake)
      nt = _popcnt(mtake)
      return o_off + nt, c_off, taken + nt
    else:
      plsc.store_compressed(dst_k.at[pl.ds(c_off, 16)], kv, mask=mc)
      plsc.store_compressed(dst_i.at[pl.ds(c_off, 16)], iv, mask=mc)
      return o_off, c_off + _popcnt(mc), taken

  out_off, c_off, _ = lax.fori_loop(0, nvreg, cbody,
                                    (out_off, jnp.int32(0), jnp.int32(0)))
  return c_off, qrem, out_off


def _sort_pass(shift, src_k, src_i, dst_k, dst_i, hist, start, iota):
  for h in range(16):
    hist[pl.ds(h * 16, 16)] = jnp.zeros((16,), jnp.int32)

  def hbody(j, carry):
    kv = src_k[pl.ds(j * 16, 16)]
    byte = lax.shift_right_logical(kv, shift) & 0xFF
    c16, last = plsc.scan_count(byte)
    plsc.addupdate_scatter(hist, [byte], c16, mask=last)
    return carry

  lax.fori_loop(0, KV, hbody, 0)

  def obody(t, cum):
    vb = 15 - t
    row = hist[pl.ds(vb * 16, 16)]
    rrow = lax.rev(row, (0,))
    incl = plsc.cumsum(rrow)
    excl = cum + incl - rrow
    start[pl.ds(vb * 16, 16)] = lax.rev(excl, (0,))
    return cum + incl[15]

  lax.fori_loop(0, 16, obody, jnp.int32(0))

  def pbody(j, carry):
    kv = src_k[pl.ds(j * 16, 16)]
    iv = src_i[pl.ds(j * 16, 16)]
    byte = lax.shift_right_logical(kv, shift) & 0xFF
    c16, last = plsc.scan_count(byte)
    base = plsc.load_gather(start, [byte])
    pos = base + c16 - 1
    plsc.store_scatter(dst_k, [pos], kv)
    plsc.store_scatter(dst_i, [pos], iv)
    plsc.addupdate_scatter(start, [byte], c16, mask=last)
    return carry

  lax.fori_loop(0, KV, pbody, 0)


def _iou_gt(cx1, cy1, cx2, cy2, car, ox1, oy1, ox2, oy2, oar):
  ltx = jnp.maximum(cx1, ox1)
  lty = jnp.maximum(cy1, oy1)
  rbx = jnp.minimum(cx2, ox2)
  rby = jnp.minimum(cy2, oy2)
  ww = jnp.maximum(rbx - ltx, 0.0)
  hh = jnp.maximum(rby - lty, 0.0)
  inter = ww * hh
  union = car + oar - inter
  return (inter / (union + 1e-9)) > NMS_T


def _rpn_body(prop, obj, shapes, boxes_out, scores_out,
              stage, key_a, idx_a, key_b, idx_b,
              out_k, out_i, out_k2, out_i2, hist, start,
              x1, y1, x2, y2, area, ssc, keep, ckb,
              fb, fs, shp, sem):
  wid = lax.axis_index("s") * 2 + lax.axis_index("c")
  img = wid
  iota = lax.iota(jnp.int32, 16)

  @pl.when(wid < NIMG)
  def _():
    pltpu.sync_copy(obj.at[img], stage)

    for z in range(OUTCAP // 16):
      out_k[pl.ds(z * 16, 16)] = jnp.zeros((16,), jnp.int32)
      out_i[pl.ds(z * 16, 16)] = jnp.zeros((16,), jnp.int32)

    cnt, q, ooff = _select_pass0(jnp.int32(K), stage, key_a, out_k, out_i,
                                 key_b, idx_b, hist, iota)
    cnt, q, ooff = _select_pass(16, cnt, q, ooff, key_b, idx_b, key_a, idx_a,
                                out_k, out_i, hist, iota, final=False)
    cnt, q, ooff = _select_pass(8, cnt, q, ooff, key_a, idx_a, key_b, idx_b,
                                out_k, out_i, hist, iota, final=False)
    cnt, q, ooff = _select_pass(0, cnt, q, ooff, key_b, idx_b, None, None,
                                out_k, out_i, hist, iota, final=True)

    _sort_pass(0, out_k, out_i, out_k2, out_i2, hist, start, iota)
    _sort_pass(8, out_k2, out_i2, out_k, out_i, hist, start, iota)
    _sort_pass(16, out_k, out_i, out_k2, out_i2, hist, start, iota)
    _sort_pass(24, out_k2, out_i2, out_k, out_i, hist, start, iota)

    planes = (x1, y1, x2, y2)
    handles = []
    for cd in range(4):
      for c in range(8):
        handles.append(pltpu.async_copy(
            prop.at[img].at[cd].at[out_i.at[pl.ds(c * 128, 128)]],
            planes[cd].at[pl.ds(c * 128, 128)], sem))
    for h in handles:
      h.wait()

    shw = plsc.load_gather(shp, [jnp.full((16,), img, jnp.int32), iota & 1])
    hf = shw[0].astype(jnp.float32)
    wf = shw[1].astype(jnp.float32)

    def vbody(v, carry):
      o = v * 16
      x1v = x1[pl.ds(o, 16)]
      y1v = y1[pl.ds(o, 16)]
      x2v = x2[pl.ds(o, 16)]
      y2v = y2[pl.ds(o, 16)]
      kv = out_k[pl.ds(v * 16, 16)]
      bb = jnp.where(kv >= 0, kv ^ NEG1, kv ^ MINI32)
      sc = plsc.bitcast(bb, jnp.float32)
      x1c = jnp.minimum(jnp.maximum(x1v, 0.0), wf)
      y1c = jnp.minimum(jnp.maximum(y1v, 0.0), hf)
      x2c = jnp.minimum(jnp.maximum(x2v, 0.0), wf)
      y2c = jnp.minimum(jnp.maximum(y2v, 0.0), hf)
      ws = x2c - x1c
      hs = y2c - y1c
      valid = (ws >= MIN_SIZE) & (hs >= MIN_SIZE) & (sc >= 0.0)
      x1[pl.ds(o, 16)] = x1c
      y1[pl.ds(o, 16)] = y1c
      x2[pl.ds(o, 16)] = x2c
      y2[pl.ds(o, 16)] = y2c
      area[pl.ds(o, 16)] = ws * hs
      ssc[pl.ds(o, 16)] = sc
      keep[pl.ds(o, 16)] = jnp.where(valid, 1, 0).astype(jnp.int32)
      return carry

    lax.fori_loop(0, KV, vbody, 0)

    def nbody(b, carry):
      base = b * 16
      bx1 = x1[pl.ds(base, 16)]
      by1 = y1[pl.ds(base, 16)]
      bx2 = x2[pl.ds(base, 16)]
      by2 = y2[pl.ds(base, 16)]
      bar = area[pl.ds(base, 16)]
      kv = keep[pl.ds(base, 16)]
      for i in range(16):
        gt = _iou_gt(bx1[i], by1[i], bx2[i], by2[i], bar[i],
                     bx1, by1, bx2, by2, bar)
        sup = gt & (iota > i) & (kv[i] != 0)
        kv = jnp.where(sup, 0, kv)
      keep[pl.ds(base, 16)] = kv
      mask = kv != 0
      zf = jnp.zeros((16,), jnp.float32)
      for r in range(5):
        ckb[pl.ds(r * 16, 16)] = zf
      plsc.store_compressed(ckb.at[pl.ds(0, 16)], bx1, mask=mask)
      plsc.store_compressed(ckb.at[pl.ds(16, 16)], by1, mask=mask)
      plsc.store_compressed(ckb.at[pl.ds(32, 16)], bx2, mask=mask)
      plsc.store_compressed(ckb.at[pl.ds(48, 16)], by2, mask=mask)
      plsc.store_compressed(ckb.at[pl.ds(64, 16)], bar, mask=mask)
      nk = _popcnt(mask)

      def tbody(g, carry2):
        boxes4 = []
        for u in range(4):
          tt = jnp.full((16,), g * 4 + u, jnp.int32)
          boxes4.append((plsc.load_gather(ckb, [tt]),
                         plsc.load_gather(ckb, [tt + 16]),
                         plsc.load_gather(ckb, [tt + 32]),
                         plsc.load_gather(ckb, [tt + 48]),
                         plsc.load_gather(ckb, [tt + 64])))

        def wbody(v, carry3):
          o = v * 16
          ox1 = x1[pl.ds(o, 16)]
          oy1 = y1[pl.ds(o, 16)]
          ox2 = x2[pl.ds(o, 16)]
          oy2 = y2[pl.ds(o, 16)]
          oar = area[pl.ds(o, 16)]
          acc = _iou_gt(*boxes4[0], ox1, oy1, ox2, oy2, oar)
          for u in range(1, 4):
            acc = acc | _iou_gt(*boxes4[u], ox1, oy1, ox2, oy2, oar)
          keep[pl.ds(o, 16)] = jnp.where(acc, 0, keep[pl.ds(o, 16)])
          return carry3

        lax.fori_loop(b + 1, KV, wbody, 0)
        return carry2

      lax.fori_loop(0, (nk + 3) >> 2, tbody, 0)
      return carry

    lax.fori_loop(0, KV, nbody, 0)

    for z in range(KV):
      fs[pl.ds(z * 16, 16)] = jnp.full((16,), -1e9, jnp.float32)
    for z in range(KP * 4 // 16):
      fb[pl.ds(z * 16, 16)] = jnp.zeros((16,), jnp.float32)

    def fbody(v, off):
      o = v * 16
      mask = keep[pl.ds(o, 16)] != 0
      pc = plsc.cumsum(jnp.where(mask, 1, 0).astype(jnp.int32))
      pos = off + pc - 1
      plsc.store_scatter(fs, [pos], ssc[pl.ds(o, 16)], mask=mask)
      plsc.store_scatter(fb, [pos * 4], x1[pl.ds(o, 16)], mask=mask)
      plsc.store_scatter(fb, [pos * 4 + 1], y1[pl.ds(o, 16)], mask=mask)
      plsc.store_scatter(fb, [pos * 4 + 2], x2[pl.ds(o, 16)], mask=mask)
      plsc.store_scatter(fb, [pos * 4 + 3], y2[pl.ds(o, 16)], mask=mask)
      return off + _popcnt(mask)

    lax.fori_loop(0, KV, fbody, jnp.int32(0))

    pltpu.sync_copy(fb.at[pl.ds(0, 4 * K)], boxes_out.at[img])
    pltpu.sync_copy(fs.at[pl.ds(0, K)], scores_out.at[img])


@functools.partial(
    pl.kernel,
    out_type=(jax.ShapeDtypeStruct((NIMG, 4 * K), jnp.float32),
              jax.ShapeDtypeStruct((NIMG, K), jnp.float32)),
    mesh=_mesh,
    compiler_params=pltpu.CompilerParams(needs_layout_passes=False,
                                         use_tc_tiling_on_sc=False),
    scratch_types=[
        pltpu.VMEM((N,), jnp.float32),
        pltpu.VMEM((N,), jnp.int32),
        pltpu.VMEM((N,), jnp.int32),
        pltpu.VMEM((N,), jnp.int32),
        pltpu.VMEM((N,), jnp.int32),
        pltpu.VMEM((OUTCAP,), jnp.int32),
        pltpu.VMEM((OUTCAP,), jnp.int32),
        pltpu.VMEM((OUTCAP,), jnp.int32),
        pltpu.VMEM((OUTCAP,), jnp.int32),
        pltpu.VMEM((256,), jnp.int32),
        pltpu.VMEM((256,), jnp.int32),
        pltpu.VMEM((OUTCAP,), jnp.float32),
        pltpu.VMEM((OUTCAP,), jnp.float32),
        pltpu.VMEM((OUTCAP,), jnp.float32),
        pltpu.VMEM((OUTCAP,), jnp.float32),
        pltpu.VMEM((KP,), jnp.float32),
        pltpu.VMEM((KP,), jnp.float32),
        pltpu.VMEM((KP,), jnp.int32),
        pltpu.VMEM((80,), jnp.float32),
        pltpu.VMEM((KP * 4,), jnp.float32),
        pltpu.VMEM((KP,), jnp.float32),
        pltpu.VMEM((NIMG, 2), jnp.int32),
        pltpu.SemaphoreType.DMA,
    ],
)
def _rpn_call(prop, obj, shapes, boxes_out, scores_out, *scratch):
  (stage, key_a, idx_a, key_b, idx_b, out_k, out_i, out_k2, out_i2,
   hist, start, x1, y1, x2, y2, area, ssc, keep, ckb, fb, fs,
   shp, sem) = scratch
  pltpu.sync_copy(shapes, shp)
  _rpn_body(prop, obj, shapes, boxes_out, scores_out,
            stage, key_a, idx_a, key_b, idx_b,
            out_k, out_i, out_k2, out_i2, hist, start,
            x1, y1, x2, y2, area, ssc, keep, ckb, fb, fs, shp, sem)


@jax.jit
def kernel(proposals, objectness, image_shapes):
  planar = jnp.transpose(proposals, (0, 2, 1))
  fb, fs = _rpn_call(planar, objectness, image_shapes)
  return fb.reshape(NIMG, K, 4), fs
